# baseline (speedup 1.0000x reference)
import jax
import jax.numpy as jnp
from jax.experimental import pallas as pl
from jax.experimental.pallas import tpu as pltpu

F_BLK = 512


def _gate_body(x_ref, w_ref, b_ref, gate_ref, mask_ref, acc_ref):
    bi = pl.program_id(0)
    fi = pl.program_id(1)
    xb = x_ref[0]
    s = (jnp.sum(xb, axis=-1) * (1.0 / 2048.0) + jnp.max(xb, axis=-1))[None, :]
    hp = jax.lax.dot_general(
        s, w_ref[...], (((1,), (1,)), ((), ())),
        preferred_element_type=jnp.float32,
    )

    @pl.when(fi == 0)
    def _init():
        acc_ref[pl.ds(bi, 1), :] = hp + b_ref[...][None, :]

    @pl.when(fi > 0)
    def _accum():
        acc_ref[pl.ds(bi, 1), :] = acc_ref[pl.ds(bi, 1), :] + hp

    last = (bi == pl.num_programs(0) - 1) & (fi == pl.num_programs(1) - 1)

    @pl.when(last)
    def _epilogue():
        h = acc_ref[...]
        h = jnp.where(h >= 0.0, h, 0.2 * h)
        iota = jax.lax.broadcasted_iota(jnp.int32, h.shape, 1)
        m1 = jnp.max(h, axis=1, keepdims=True)
        i1 = jnp.min(jnp.where(h == m1, iota, 16), axis=1, keepdims=True)
        h2 = jnp.where(iota == i1, -jnp.inf, h)
        m2 = jnp.max(h2, axis=1, keepdims=True)
        i2 = jnp.min(jnp.where(h2 == m2, iota, 16), axis=1, keepdims=True)
        sel = (iota == i1) | (iota == i2)
        mask_ref[...] = sel.astype(jnp.float32)
        d = jnp.where(sel, jnp.exp(h - m1), 0.0)
        gate_ref[...] = d / jnp.sum(d, axis=1, keepdims=True)


def kernel(x, W, b):
    B, F, C = x.shape
    E = W.shape[0]
    grid = (B, F // F_BLK)
    gating, mask = pl.pallas_call(
        _gate_body,
        grid=grid,
        in_specs=[
            pl.BlockSpec((1, F_BLK, C), lambda b, f: (b, f, 0)),
            pl.BlockSpec((E, F_BLK), lambda b, f: (0, f)),
            pl.BlockSpec((E,), lambda b, f: (0,)),
        ],
        out_specs=[
            pl.BlockSpec((B, E), lambda b, f: (0, 0)),
            pl.BlockSpec((B, E), lambda b, f: (0, 0)),
        ],
        out_shape=[
            jax.ShapeDtypeStruct((B, E), jnp.float32),
            jax.ShapeDtypeStruct((B, E), jnp.float32),
        ],
        scratch_shapes=[pltpu.VMEM((B, E), jnp.float32)],
    )(x, W, b)
    return gating, mask


# speedup vs baseline: 1.1384x; 1.1384x over previous
import jax
import jax.numpy as jnp
from jax.experimental import pallas as pl
from jax.experimental.pallas import tpu as pltpu

F_BLK = 1024


def _gate_body(x_ref, w_ref, b_ref, gate_ref, mask_ref, acc_ref):
    bi = pl.program_id(0)
    fi = pl.program_id(1)
    xb = x_ref[0]
    s = (jnp.sum(xb, axis=-1) * (1.0 / 2048.0) + jnp.max(xb, axis=-1))[None, :]
    hp = jax.lax.dot_general(
        s, w_ref[...], (((1,), (1,)), ((), ())),
        preferred_element_type=jnp.float32,
    )

    @pl.when(fi == 0)
    def _init():
        acc_ref[pl.ds(bi, 1), :] = hp + b_ref[...][None, :]

    @pl.when(fi > 0)
    def _accum():
        acc_ref[pl.ds(bi, 1), :] = acc_ref[pl.ds(bi, 1), :] + hp

    last = (bi == pl.num_programs(0) - 1) & (fi == pl.num_programs(1) - 1)

    @pl.when(last)
    def _epilogue():
        h = acc_ref[...]
        h = jnp.where(h >= 0.0, h, 0.2 * h)
        iota = jax.lax.broadcasted_iota(jnp.int32, h.shape, 1)
        m1 = jnp.max(h, axis=1, keepdims=True)
        i1 = jnp.min(jnp.where(h == m1, iota, 16), axis=1, keepdims=True)
        h2 = jnp.where(iota == i1, -jnp.inf, h)
        m2 = jnp.max(h2, axis=1, keepdims=True)
        i2 = jnp.min(jnp.where(h2 == m2, iota, 16), axis=1, keepdims=True)
        sel = (iota == i1) | (iota == i2)
        mask_ref[...] = sel.astype(jnp.float32)
        d = jnp.where(sel, jnp.exp(h - m1), 0.0)
        gate_ref[...] = d / jnp.sum(d, axis=1, keepdims=True)


def kernel(x, W, b):
    B, F, C = x.shape
    E = W.shape[0]
    grid = (B, F // F_BLK)
    gating, mask = pl.pallas_call(
        _gate_body,
        grid=grid,
        in_specs=[
            pl.BlockSpec((1, F_BLK, C), lambda b, f: (b, f, 0)),
            pl.BlockSpec((E, F_BLK), lambda b, f: (0, f)),
            pl.BlockSpec((E,), lambda b, f: (0,)),
        ],
        out_specs=[
            pl.BlockSpec((B, E), lambda b, f: (0, 0)),
            pl.BlockSpec((B, E), lambda b, f: (0, 0)),
        ],
        out_shape=[
            jax.ShapeDtypeStruct((B, E), jnp.float32),
            jax.ShapeDtypeStruct((B, E), jnp.float32),
        ],
        scratch_shapes=[pltpu.VMEM((B, E), jnp.float32)],
    )(x, W, b)
    return gating, mask
